# baseline (speedup 1.0000x reference)
import functools

import jax
import jax.numpy as jnp
import numpy as np
from jax.experimental import pallas as pl
from jax.experimental.pallas import tpu as pltpu

N = 10000
E = 320000
C = 128
NUM_RBF = 32
SH_DIM = 8
N_DEG = 2
H = 4
DH = C // H
NUM_LAYERS = 2
CUTOFF = 5.0
AVG_NEIGH = 32.0
SPHC_NORM = 32.0

BE = 2560


def _silu(x):
    return x * jax.nn.sigmoid(x)


def _sph_harm(vec):
    v = vec / (jnp.linalg.norm(vec, axis=-1, keepdims=True) + 1e-9)
    x, y, z = v[:, 0], v[:, 1], v[:, 2]
    c1 = jnp.sqrt(3.0)
    y1 = jnp.stack([c1 * x, c1 * y, c1 * z], axis=-1)
    c2 = jnp.sqrt(15.0)
    y2 = jnp.stack([c2 * x * y, c2 * y * z,
                    (jnp.sqrt(5.0) / 2.0) * (3.0 * z * z - 1.0),
                    c2 * x * z, (c2 / 2.0) * (x * x - y * y)], axis=-1)
    return jnp.concatenate([y1, y2], axis=-1)


def _rbf(r):
    mu = jnp.linspace(jnp.exp(-CUTOFF), 1.0, NUM_RBF)
    beta = (2.0 / NUM_RBF * (1.0 - jnp.exp(-CUTOFF))) ** (-2)
    return jnp.exp(-beta * (jnp.exp(-r)[:, None] - mu[None, :]) ** 2)


def _expand_deg(w):
    return jnp.concatenate([jnp.repeat(w[..., 0:1], 3, axis=-1),
                            jnp.repeat(w[..., 1:2], 5, axis=-1)], axis=-1)


def _deg_norm(a):
    n1 = jnp.sqrt(jnp.sum(a[..., 0:3] ** 2, axis=-1) + 1e-9)
    n2 = jnp.sqrt(jnp.sum(a[..., 3:8] ** 2, axis=-1) + 1e-9)
    return jnp.stack([n1, n2], axis=-1)


def _filter_body(ef_ref, cs_ref, frw1, frw2, fsw1, fsw2, grw1, grw2,
                 gsw1, gsw2, w_ref, g_ref):
    ef = ef_ref[...]
    cs = cs_ref[...]
    h1 = _silu(jnp.dot(ef, frw1[...], preferred_element_type=jnp.float32))
    h2 = _silu(jnp.dot(cs, fsw1[...], preferred_element_type=jnp.float32))
    w_ref[...] = (jnp.dot(h1, frw2[...], preferred_element_type=jnp.float32)
                  + jnp.dot(h2, fsw2[...], preferred_element_type=jnp.float32))
    g1 = _silu(jnp.dot(ef, grw1[...], preferred_element_type=jnp.float32))
    g2 = _silu(jnp.dot(cs, gsw1[...], preferred_element_type=jnp.float32))
    g_ref[...] = (jnp.dot(g1, grw2[...], preferred_element_type=jnp.float32)
                  + jnp.dot(g2, gsw2[...], preferred_element_type=jnp.float32))


@jax.jit
def _filter_call(ef, cs, frw1, frw2, fsw1, fsw2, grw1, grw2, gsw1, gsw2):
    grid = (E // BE,)
    w, g = pl.pallas_call(
        _filter_body,
        grid=grid,
        in_specs=[
            pl.BlockSpec((BE, NUM_RBF), lambda i: (i, 0)),
            pl.BlockSpec((BE, N_DEG), lambda i: (i, 0)),
            pl.BlockSpec((NUM_RBF, C), lambda i: (0, 0)),
            pl.BlockSpec((C, C), lambda i: (0, 0)),
            pl.BlockSpec((N_DEG, 32), lambda i: (0, 0)),
            pl.BlockSpec((32, C), lambda i: (0, 0)),
            pl.BlockSpec((NUM_RBF, C), lambda i: (0, 0)),
            pl.BlockSpec((C, N_DEG), lambda i: (0, 0)),
            pl.BlockSpec((N_DEG, 32), lambda i: (0, 0)),
            pl.BlockSpec((32, N_DEG), lambda i: (0, 0)),
        ],
        out_specs=[
            pl.BlockSpec((BE, C), lambda i: (i, 0)),
            pl.BlockSpec((BE, N_DEG), lambda i: (i, 0)),
        ],
        out_shape=[
            jax.ShapeDtypeStruct((E, C), jnp.float32),
            jax.ShapeDtypeStruct((E, N_DEG), jnp.float32),
        ],
    )(ef, cs, frw1, frw2, fsw1, fsw2, grw1, grw2, gsw1, gsw2)
    return w, g


def kernel(edge_vectors, distances, cutoffs, params, node_species, senders,
           receivers):
    ef = _rbf(distances)
    sh = _sph_harm(edge_vectors)
    shc = sh * cutoffs[:, None]
    x = params['embed'][node_species]
    chi = jax.ops.segment_sum(shc, receivers, num_segments=N) / SPHC_NORM
    for l in range(NUM_LAYERS):
        p = params['layers'][l]
        chi_ij = chi[senders] - chi[receivers]
        cs = _deg_norm(chi_ij)
        w, g = _filter_call(ef, cs, p['frw1'], p['frw2'], p['fsw1'],
                            p['fsw2'], p['grw1'], p['grw2'], p['gsw1'],
                            p['gsw2'])
        q = (x @ p['wq']).reshape(N, H, DH)
        k = (x @ p['wk']).reshape(N, H, DH)
        v = (x @ p['wv']).reshape(N, H, DH)
        wr = w.reshape(E, H, DH)
        alpha = jnp.sum(q[receivers] * wr * k[senders], axis=-1) / np.sqrt(DH)
        alpha = alpha * (cutoffs[:, None] / AVG_NEIGH)
        msg = (alpha[..., None] * v[senders]).reshape(E, C)
        x = x + jax.ops.segment_sum(msg, receivers, num_segments=N)
        gw = g * (x @ p['gn'])[senders]
        dchi = jax.ops.segment_sum(shc * _expand_deg(gw), receivers,
                                   num_segments=N) / AVG_NEIGH
        chi = chi + dchi
        y = jnp.concatenate([x, _deg_norm(chi)], axis=-1) @ p['ib']
        x = x + y[:, :C]
        chi = chi + chi * _expand_deg(y[:, C:])
    e = _silu(x @ params['m1']) @ params['m2']
    return e.squeeze(axis=-1)


# speedup vs baseline: 1.0900x; 1.0900x over previous
import functools

import jax
import jax.numpy as jnp
import numpy as np
from jax import lax
from jax.experimental import pallas as pl
from jax.experimental.pallas import tpu as pltpu
from jax.experimental.pallas import tpu_sc as plsc

N = 10000
E = 320000
C = 128
NUM_RBF = 32
SH_DIM = 8
N_DEG = 2
H = 4
DH = C // H
NUM_LAYERS = 2
CUTOFF = 5.0
AVG_NEIGH = 32.0
SPHC_NORM = 32.0

BE = 2560


def _silu(x):
    return x * jax.nn.sigmoid(x)


def _sph_harm(vec):
    v = vec / (jnp.linalg.norm(vec, axis=-1, keepdims=True) + 1e-9)
    x, y, z = v[:, 0], v[:, 1], v[:, 2]
    c1 = jnp.sqrt(3.0)
    y1 = jnp.stack([c1 * x, c1 * y, c1 * z], axis=-1)
    c2 = jnp.sqrt(15.0)
    y2 = jnp.stack([c2 * x * y, c2 * y * z,
                    (jnp.sqrt(5.0) / 2.0) * (3.0 * z * z - 1.0),
                    c2 * x * z, (c2 / 2.0) * (x * x - y * y)], axis=-1)
    return jnp.concatenate([y1, y2], axis=-1)


def _rbf(r):
    mu = jnp.linspace(jnp.exp(-CUTOFF), 1.0, NUM_RBF)
    beta = (2.0 / NUM_RBF * (1.0 - jnp.exp(-CUTOFF))) ** (-2)
    return jnp.exp(-beta * (jnp.exp(-r)[:, None] - mu[None, :]) ** 2)


def _expand_deg(w):
    return jnp.concatenate([jnp.repeat(w[..., 0:1], 3, axis=-1),
                            jnp.repeat(w[..., 1:2], 5, axis=-1)], axis=-1)


def _deg_norm(a):
    n1 = jnp.sqrt(jnp.sum(a[..., 0:3] ** 2, axis=-1) + 1e-9)
    n2 = jnp.sqrt(jnp.sum(a[..., 3:8] ** 2, axis=-1) + 1e-9)
    return jnp.stack([n1, n2], axis=-1)


def _filter_body(ef_ref, cs_ref, frw1, frw2, fsw1, fsw2, grw1, grw2,
                 gsw1, gsw2, w_ref, g_ref):
    ef = ef_ref[...]
    cs = cs_ref[...]
    h1 = _silu(jnp.dot(ef, frw1[...], preferred_element_type=jnp.float32))
    h2 = _silu(jnp.dot(cs, fsw1[...], preferred_element_type=jnp.float32))
    w_ref[...] = (jnp.dot(h1, frw2[...], preferred_element_type=jnp.float32)
                  + jnp.dot(h2, fsw2[...], preferred_element_type=jnp.float32))
    g1 = _silu(jnp.dot(ef, grw1[...], preferred_element_type=jnp.float32))
    g2 = _silu(jnp.dot(cs, gsw1[...], preferred_element_type=jnp.float32))
    g_ref[...] = (jnp.dot(g1, grw2[...], preferred_element_type=jnp.float32)
                  + jnp.dot(g2, gsw2[...], preferred_element_type=jnp.float32))


@jax.jit
def _filter_call(ef, cs, frw1, frw2, fsw1, fsw2, grw1, grw2, gsw1, gsw2):
    grid = (E // BE,)
    w, g = pl.pallas_call(
        _filter_body,
        grid=grid,
        in_specs=[
            pl.BlockSpec((BE, NUM_RBF), lambda i: (i, 0)),
            pl.BlockSpec((BE, N_DEG), lambda i: (i, 0)),
            pl.BlockSpec((NUM_RBF, C), lambda i: (0, 0)),
            pl.BlockSpec((C, C), lambda i: (0, 0)),
            pl.BlockSpec((N_DEG, 32), lambda i: (0, 0)),
            pl.BlockSpec((32, C), lambda i: (0, 0)),
            pl.BlockSpec((NUM_RBF, C), lambda i: (0, 0)),
            pl.BlockSpec((C, N_DEG), lambda i: (0, 0)),
            pl.BlockSpec((N_DEG, 32), lambda i: (0, 0)),
            pl.BlockSpec((32, N_DEG), lambda i: (0, 0)),
        ],
        out_specs=[
            pl.BlockSpec((BE, C), lambda i: (i, 0)),
            pl.BlockSpec((BE, N_DEG), lambda i: (i, 0)),
        ],
        out_shape=[
            jax.ShapeDtypeStruct((E, C), jnp.float32),
            jax.ShapeDtypeStruct((E, N_DEG), jnp.float32),
        ],
    )(ef, cs, frw1, frw2, fsw1, fsw2, grw1, grw2, gsw1, gsw2)
    return w, g


_SC_MESH = plsc.VectorSubcoreMesh(core_axis_name="c", subcore_axis_name="s")
_NW = 32
_EW = E // _NW
_KCH = 80
_NCH = _EW // _KCH
_NPAD = 10240
_NR = _NPAD // 16


def _segsum_body(data_hbm, recv_hbm, zeros_hbm, out_hbm, idx_v, dbuf, acc):
    cid = lax.axis_index("c")
    sid = lax.axis_index("s")
    wid = cid * 16 + sid
    base = wid * _EW
    rbase = sid * _NR
    pltpu.sync_copy(zeros_hbm.at[pl.ds(rbase, _NR), :],
                    acc.at[pl.ds(rbase, _NR), :])
    plsc.subcore_barrier()

    def chunk(j, carry):
        s = base + j * _KCH
        pltpu.sync_copy(recv_hbm.at[pl.ds(s, _KCH)], idx_v)
        pltpu.sync_copy(data_hbm.at[pl.ds(s, _KCH), :], dbuf)
        pltpu.sync_copy(dbuf, acc.at[idx_v], add=True)
        return carry

    lax.fori_loop(0, _NCH, chunk, 0)
    plsc.subcore_barrier()
    pltpu.sync_copy(acc.at[pl.ds(rbase, _NR), :],
                    out_hbm.at[cid, pl.ds(rbase, _NR), :])


def _make_segsum(d):
    return pl.kernel(
        _segsum_body,
        out_type=jax.ShapeDtypeStruct((2, _NPAD, d), jnp.float32),
        mesh=_SC_MESH,
        scratch_types=[
            pltpu.VMEM((_KCH,), jnp.int32),
            pltpu.VMEM((_KCH, d), jnp.float32),
            pltpu.VMEM_SHARED((_NPAD, d), jnp.float32),
        ],
    )


_segsum128 = _make_segsum(128)


def _sc_segment_sum(data, receivers):
    d = data.shape[-1]
    if d < 128:
        data = jnp.pad(data, ((0, 0), (0, 128 - d)))
    p = _segsum128(data, receivers, jnp.zeros((_NPAD, 128), jnp.float32))
    return (p[0] + p[1])[:N, :d]


def kernel(edge_vectors, distances, cutoffs, params, node_species, senders,
           receivers):
    ef = _rbf(distances)
    sh = _sph_harm(edge_vectors)
    shc = sh * cutoffs[:, None]
    x = params['embed'][node_species]
    chi = _sc_segment_sum(shc, receivers) / SPHC_NORM
    for l in range(NUM_LAYERS):
        p = params['layers'][l]
        chi_ij = chi[senders] - chi[receivers]
        cs = _deg_norm(chi_ij)
        w, g = _filter_call(ef, cs, p['frw1'], p['frw2'], p['fsw1'],
                            p['fsw2'], p['grw1'], p['grw2'], p['gsw1'],
                            p['gsw2'])
        q = (x @ p['wq']).reshape(N, H, DH)
        k = (x @ p['wk']).reshape(N, H, DH)
        v = (x @ p['wv']).reshape(N, H, DH)
        wr = w.reshape(E, H, DH)
        alpha = jnp.sum(q[receivers] * wr * k[senders], axis=-1) / np.sqrt(DH)
        alpha = alpha * (cutoffs[:, None] / AVG_NEIGH)
        msg = (alpha[..., None] * v[senders]).reshape(E, C)
        x = x + _sc_segment_sum(msg, receivers)
        gw = g * (x @ p['gn'])[senders]
        dchi = _sc_segment_sum(shc * _expand_deg(gw), receivers) / AVG_NEIGH
        chi = chi + dchi
        y = jnp.concatenate([x, _deg_norm(chi)], axis=-1) @ p['ib']
        x = x + y[:, :C]
        chi = chi + chi * _expand_deg(y[:, C:])
    e = _silu(x @ params['m1']) @ params['m2']
    return e.squeeze(axis=-1)
